# baseline (speedup 1.0000x reference)
import functools

import jax
import jax.numpy as jnp
from jax import lax
from jax.experimental import pallas as pl
from jax.experimental.pallas import tpu as pltpu
from jax.experimental.pallas import tpu_sc as plsc

N_NODES = 10000
N_EDGES = 160000
MUL = 16
NUM_BASIS = 32
MAX_RADIUS = 2.5
NUM_NEIGHBORS = 16
HIDDEN = 32
SILU_NORM = 1.6790

E_PAD = 163840
EDGE_BLK = 2048

_NW = 32
_BE = 128
_NB = N_EDGES // _BE
_MAXB = (_NB + _NW - 1) // _NW

_SC_MESH = plsc.VectorSubcoreMesh(core_axis_name="c", subcore_axis_name="s")


_U = 8
_FULL = (_NB // _NW) // _U


def _sc_gather_body(x_hbm, pos_hbm, src_hbm, dst_hbm, xsrc_out, sq_out,
                    pos_v, idx_s, idx_d, rows, sq_v, semi, semg, semo):
    w = lax.axis_index("s") * 2 + lax.axis_index("c")
    pltpu.sync_copy(pos_hbm, pos_v)

    def compute_sq(u, idx_s_u, idx_d_u, sq_u):
        for i in range(_BE // 16):
            si = idx_s_u[pl.ds(i * 16, 16)] * 3
            di = idx_d_u[pl.ds(i * 16, 16)] * 3
            dx = plsc.load_gather(pos_v, [si]) - plsc.load_gather(pos_v, [di])
            dy = plsc.load_gather(pos_v, [si + 1]) - plsc.load_gather(pos_v, [di + 1])
            dz = plsc.load_gather(pos_v, [si + 2]) - plsc.load_gather(pos_v, [di + 2])
            sq_u[pl.ds(i * 16, 16)] = dx * dx + dy * dy + dz * dz

    def body(j, carry):
        bases = [(w + _NW * (_U * j + u)) * _BE for u in range(_U)]
        cps = []
        for u, base in enumerate(bases):
            cps.append(pltpu.async_copy(src_hbm.at[pl.ds(base, _BE)], idx_s.at[u], semi))
            cps.append(pltpu.async_copy(dst_hbm.at[pl.ds(base, _BE)], idx_d.at[u], semi))
        for cp in cps:
            cp.wait()
        gs = [pltpu.async_copy(x_hbm.at[idx_s.at[u]], rows.at[u], semg)
              for u in range(_U)]
        for u in range(_U):
            compute_sq(u, idx_s.at[u], idx_d.at[u], sq_v.at[u])
        os = []
        for u, base in enumerate(bases):
            gs[u].wait()
            os.append(pltpu.async_copy(rows.at[u], xsrc_out.at[pl.ds(base, _BE)], semo))
            os.append(pltpu.async_copy(sq_v.at[u], sq_out.at[pl.ds(base, _BE)], semo))
        for cp in os:
            cp.wait()
        return carry

    lax.fori_loop(0, _FULL, body, 0)

    def tail(j, carry):
        b = w + _NW * j

        @pl.when(b < _NB)
        def _():
            base = b * _BE
            pltpu.sync_copy(src_hbm.at[pl.ds(base, _BE)], idx_s.at[0])
            pltpu.sync_copy(dst_hbm.at[pl.ds(base, _BE)], idx_d.at[0])
            cp = pltpu.async_copy(x_hbm.at[idx_s.at[0]], rows.at[0], semg)
            compute_sq(0, idx_s.at[0], idx_d.at[0], sq_v.at[0])
            cp.wait()
            pltpu.sync_copy(rows.at[0], xsrc_out.at[pl.ds(base, _BE)])
            pltpu.sync_copy(sq_v.at[0], sq_out.at[pl.ds(base, _BE)])

        return carry

    lax.fori_loop(_FULL * _U, _MAXB, tail, 0)


@functools.partial(
    pl.kernel,
    mesh=_SC_MESH,
    compiler_params=pltpu.CompilerParams(
        needs_layout_passes=False, use_tc_tiling_on_sc=False),
    out_type=[
        jax.ShapeDtypeStruct((E_PAD, MUL), jnp.float32),
        jax.ShapeDtypeStruct((E_PAD,), jnp.float32),
    ],
    scratch_types=[
        pltpu.VMEM((N_NODES * 3,), jnp.float32),
        pltpu.VMEM((_U, _BE), jnp.int32),
        pltpu.VMEM((_U, _BE), jnp.int32),
        pltpu.VMEM((_U, _BE, MUL), jnp.float32),
        pltpu.VMEM((_U, _BE), jnp.float32),
        pltpu.SemaphoreType.DMA,
        pltpu.SemaphoreType.DMA,
        pltpu.SemaphoreType.DMA,
    ],
)
def _sc_gather(x_hbm, pos_hbm, src_hbm, dst_hbm, xsrc_out, sq_out,
               pos_v, idx_s, idx_d, rows, sq_v, semi, semg, semo):
    _sc_gather_body(x_hbm, pos_hbm, src_hbm, dst_hbm, xsrc_out, sq_out,
                    pos_v, idx_s, idx_d, rows, sq_v, semi, semg, semo)


_N_PAD = 10240
_SL = _N_PAD // 16
_SC_PARAMS = pltpu.CompilerParams(
    needs_layout_passes=False, use_tc_tiling_on_sc=False)
_IOTA16 = None


def _iota16():
    return jnp.arange(16, dtype=jnp.int32)


def _worker_id():
    return lax.axis_index("s") * 2 + lax.axis_index("c")


def _combine_dups(sk, sv, kb, vb, is_max):
    iota = _iota16()
    kb[...] = sk
    vb[...] = sv
    for d in (1, 2, 4, 8):
        g = jnp.maximum(iota - d, 0)
        ks = plsc.load_gather(kb, [g])
        vs = plsc.load_gather(vb, [g])
        comb = jnp.maximum(sv, vs) if is_max else sv + vs
        sv = jnp.where(jnp.logical_and(ks == sk, iota >= d), comb, sv)
        vb[...] = sv
    kn = plsc.load_gather(kb, [jnp.minimum(iota + 1, 15)])
    is_last = jnp.logical_or(sk != kn, iota == 15)
    return sv, is_last


def _spmem_combine(acc_v, sh, part_hbm, buf, is_max):
    sid = lax.axis_index("s")
    cid = lax.axis_index("c")
    pltpu.sync_copy(acc_v, sh.at[sid])
    plsc.subcore_barrier()
    pltpu.sync_copy(sh.at[:, pl.ds(sid * _SL, _SL)], buf)

    def red(j, c):
        o = j * 16
        v = buf[0, pl.ds(o, 16)]
        for k in range(1, 16):
            vk = buf[k, pl.ds(o, 16)]
            v = jnp.maximum(v, vk) if is_max else v + vk
        acc_v[pl.ds(o, 16)] = v
        return c

    lax.fori_loop(0, _SL // 16, red, 0)
    pltpu.sync_copy(acc_v.at[pl.ds(0, _SL)], part_hbm.at[cid, pl.ds(sid * _SL, _SL)])


@functools.partial(
    pl.kernel,
    mesh=_SC_MESH,
    compiler_params=_SC_PARAMS,
    out_type=[jax.ShapeDtypeStruct((2, _N_PAD), jnp.float32)],
    scratch_types=[
        pltpu.VMEM((_N_PAD,), jnp.float32),
        pltpu.VMEM((_U, _BE), jnp.int32),
        pltpu.VMEM((_U, _BE), jnp.float32),
        pltpu.VMEM((16,), jnp.int32),
        pltpu.VMEM((16,), jnp.float32),
        pltpu.VMEM((16, _SL), jnp.float32),
        pltpu.VMEM_SHARED((16, _N_PAD), jnp.float32),
        pltpu.SemaphoreType.DMA,
    ],
)
def _sc_segmax(src_hbm, logits_hbm, m_p, m_t, idx_v, val_v, kb, vb, buf, sh, semi):
    w = _worker_id()
    neg = jnp.full((16,), -3.0e38, jnp.float32)

    def initb(i, c):
        m_t[pl.ds(i * 16, 16)] = neg
        return c

    lax.fori_loop(0, _N_PAD // 16, initb, 0)

    def blk(u):
        for i in range(_BE // 16):
            k = idx_v.at[u][pl.ds(i * 16, 16)]
            v = val_v.at[u][pl.ds(i * 16, 16)]
            sk, sv = plsc.sort_key_val(k, v)
            sv, is_last = _combine_dups(sk, sv, kb, vb, True)
            cur = plsc.load_gather(m_t, [sk])
            plsc.store_scatter(m_t, [sk], jnp.maximum(cur, sv), mask=is_last)

    def body(j, c):
        bases = [(w + _NW * (_U * j + u)) * _BE for u in range(_U)]
        cps = []
        for u, base in enumerate(bases):
            cps.append(pltpu.async_copy(src_hbm.at[pl.ds(base, _BE)], idx_v.at[u], semi))
            cps.append(pltpu.async_copy(logits_hbm.at[pl.ds(base, _BE)], val_v.at[u], semi))
        for cp in cps:
            cp.wait()
        for u in range(_U):
            blk(u)
        return c

    lax.fori_loop(0, _FULL, body, 0)

    def tail(j, c):
        b = w + _NW * j

        @pl.when(b < _NB)
        def _():
            base = b * _BE
            pltpu.sync_copy(src_hbm.at[pl.ds(base, _BE)], idx_v.at[0])
            pltpu.sync_copy(logits_hbm.at[pl.ds(base, _BE)], val_v.at[0])
            blk(0)

        return c

    lax.fori_loop(_FULL * _U, _MAXB, tail, 0)
    _spmem_combine(m_t, sh, m_p, buf, True)


@functools.partial(
    pl.kernel,
    mesh=_SC_MESH,
    compiler_params=_SC_PARAMS,
    out_type=[
        jax.ShapeDtypeStruct((E_PAD,), jnp.float32),
        jax.ShapeDtypeStruct((2, _N_PAD), jnp.float32),
    ],
    scratch_types=[
        pltpu.VMEM((_N_PAD,), jnp.float32),
        pltpu.VMEM((_N_PAD,), jnp.float32),
        pltpu.VMEM((_N_PAD,), jnp.float32),
        pltpu.VMEM((_U, _BE), jnp.int32),
        pltpu.VMEM((_U, _BE), jnp.float32),
        pltpu.VMEM((_U, _BE), jnp.float32),
        pltpu.VMEM((16,), jnp.int32),
        pltpu.VMEM((16,), jnp.float32),
        pltpu.VMEM((16, _SL), jnp.float32),
        pltpu.VMEM_SHARED((16, _N_PAD), jnp.float32),
        pltpu.SemaphoreType.DMA,
        pltpu.SemaphoreType.DMA,
    ],
)
def _sc_segsum(src_hbm, logits_hbm, m_p_hbm, exh_out, s_p,
               ma, mb, s_t, idx_v, val_v, eh_v, kb, vb, buf, sh, semi, semo):
    w = _worker_id()
    pltpu.sync_copy(m_p_hbm.at[0], ma)
    pltpu.sync_copy(m_p_hbm.at[1], mb)

    def mmax(i, c):
        o = i * 16
        ma[pl.ds(o, 16)] = jnp.maximum(ma[pl.ds(o, 16)], mb[pl.ds(o, 16)])
        return c

    lax.fori_loop(0, _N_PAD // 16, mmax, 0)

    zv = jnp.zeros((16,), jnp.float32)

    def initb(i, c):
        s_t[pl.ds(i * 16, 16)] = zv
        return c

    lax.fori_loop(0, _N_PAD // 16, initb, 0)

    def blk(u):
        for i in range(_BE // 16):
            k = idx_v.at[u][pl.ds(i * 16, 16)]
            lg = val_v.at[u][pl.ds(i * 16, 16)]
            mg = plsc.load_gather(ma, [k])
            eh = jnp.exp(0.5 * (lg - mg))
            eh_v.at[u][pl.ds(i * 16, 16)] = eh
            sk, sv = plsc.sort_key_val(k, eh * eh)
            sv, is_last = _combine_dups(sk, sv, kb, vb, False)
            cur = plsc.load_gather(s_t, [sk])
            plsc.store_scatter(s_t, [sk], cur + sv, mask=is_last)

    def body(j, c):
        bases = [(w + _NW * (_U * j + u)) * _BE for u in range(_U)]
        cps = []
        for u, base in enumerate(bases):
            cps.append(pltpu.async_copy(src_hbm.at[pl.ds(base, _BE)], idx_v.at[u], semi))
            cps.append(pltpu.async_copy(logits_hbm.at[pl.ds(base, _BE)], val_v.at[u], semi))
        for cp in cps:
            cp.wait()
        os = []
        for u, base in enumerate(bases):
            blk(u)
            os.append(pltpu.async_copy(eh_v.at[u], exh_out.at[pl.ds(base, _BE)], semo))
        for cp in os:
            cp.wait()
        return c

    lax.fori_loop(0, _FULL, body, 0)

    def tail(j, c):
        b = w + _NW * j

        @pl.when(b < _NB)
        def _():
            base = b * _BE
            pltpu.sync_copy(src_hbm.at[pl.ds(base, _BE)], idx_v.at[0])
            pltpu.sync_copy(logits_hbm.at[pl.ds(base, _BE)], val_v.at[0])
            blk(0)
            pltpu.sync_copy(eh_v.at[0], exh_out.at[pl.ds(base, _BE)])

        return c

    lax.fori_loop(_FULL * _U, _MAXB, tail, 0)
    _spmem_combine(s_t, sh, s_p, buf, False)


@functools.partial(
    pl.kernel,
    mesh=_SC_MESH,
    compiler_params=_SC_PARAMS,
    out_type=[jax.ShapeDtypeStruct((E_PAD,), jnp.float32)],
    scratch_types=[
        pltpu.VMEM((_N_PAD,), jnp.float32),
        pltpu.VMEM((_N_PAD,), jnp.float32),
        pltpu.VMEM((_U, _BE), jnp.int32),
        pltpu.VMEM((_U, _BE), jnp.float32),
        pltpu.SemaphoreType.DMA,
        pltpu.SemaphoreType.DMA,
    ],
)
def _sc_gather_s(src_hbm, s_p_hbm, ssrc_out, sa, sb, idx_v, sg_v, semi, semo):
    w = _worker_id()
    pltpu.sync_copy(s_p_hbm.at[0], sa)
    pltpu.sync_copy(s_p_hbm.at[1], sb)

    def madd(i, c):
        o = i * 16
        sa[pl.ds(o, 16)] = sa[pl.ds(o, 16)] + sb[pl.ds(o, 16)]
        return c

    lax.fori_loop(0, _N_PAD // 16, madd, 0)

    def blk(u):
        for i in range(_BE // 16):
            k = idx_v.at[u][pl.ds(i * 16, 16)]
            sg_v.at[u][pl.ds(i * 16, 16)] = plsc.load_gather(sa, [k])

    def body(j, c):
        bases = [(w + _NW * (_U * j + u)) * _BE for u in range(_U)]
        cps = [pltpu.async_copy(src_hbm.at[pl.ds(base, _BE)], idx_v.at[u], semi)
               for u, base in enumerate(bases)]
        for cp in cps:
            cp.wait()
        os = []
        for u, base in enumerate(bases):
            blk(u)
            os.append(pltpu.async_copy(sg_v.at[u], ssrc_out.at[pl.ds(base, _BE)], semo))
        for cp in os:
            cp.wait()
        return c

    lax.fori_loop(0, _FULL, body, 0)

    def tail(j, c):
        b = w + _NW * j

        @pl.when(b < _NB)
        def _():
            base = b * _BE
            pltpu.sync_copy(src_hbm.at[pl.ds(base, _BE)], idx_v.at[0])
            blk(0)
            pltpu.sync_copy(sg_v.at[0], ssrc_out.at[pl.ds(base, _BE)])

        return c

    lax.fori_loop(_FULL * _U, _MAXB, tail, 0)


@functools.partial(
    pl.kernel,
    mesh=_SC_MESH,
    compiler_params=_SC_PARAMS,
    out_type=[jax.ShapeDtypeStruct((2, _N_PAD, MUL), jnp.float32)],
    scratch_types=[
        pltpu.VMEM((_U, _BE), jnp.int32),
        pltpu.VMEM((_U, _BE, MUL), jnp.float32),
        pltpu.VMEM_SHARED((_N_PAD, MUL), jnp.float32),
        pltpu.SemaphoreType.DMA,
    ],
)
def _sc_scatter_out(dst_hbm, scaled_hbm, zeros_hbm, out_p, idx_v, rows_v, oacc,
                    semi):
    w = _worker_id()
    sid = lax.axis_index("s")
    cid = lax.axis_index("c")
    pltpu.sync_copy(zeros_hbm, oacc.at[pl.ds(sid * _SL, _SL)])
    plsc.subcore_barrier()

    def body(j, c):
        bases = [(w + _NW * (_U * j + u)) * _BE for u in range(_U)]
        cps = []
        for u, base in enumerate(bases):
            cps.append(pltpu.async_copy(dst_hbm.at[pl.ds(base, _BE)], idx_v.at[u], semi))
            cps.append(pltpu.async_copy(scaled_hbm.at[pl.ds(base, _BE)], rows_v.at[u], semi))
        for cp in cps:
            cp.wait()
        for u in range(_U):
            pltpu.sync_copy(rows_v.at[u], oacc.at[idx_v.at[u]], add=True)
        return c

    lax.fori_loop(0, _FULL, body, 0)

    def tail(j, c):
        b = w + _NW * j

        @pl.when(b < _NB)
        def _():
            base = b * _BE
            pltpu.sync_copy(dst_hbm.at[pl.ds(base, _BE)], idx_v.at[0])
            pltpu.sync_copy(scaled_hbm.at[pl.ds(base, _BE)], rows_v.at[0])
            pltpu.sync_copy(rows_v.at[0], oacc.at[idx_v.at[0]], add=True)

        return c

    lax.fori_loop(_FULL * _U, _MAXB, tail, 0)
    plsc.subcore_barrier()
    pltpu.sync_copy(oacc.at[pl.ds(sid * _SL, _SL)],
                    out_p.at[cid, pl.ds(sid * _SL, _SL)])


def _scale_body(vT_ref, eh_ref, ss_ref, o_ref):
    wgt = eh_ref[...] * jax.lax.rsqrt(ss_ref[...])
    o_ref[...] = (vT_ref[...] * wgt).T


def _scale_values(valuesT, exh, ssrc):
    grid = E_PAD // EDGE_BLK
    return pl.pallas_call(
        _scale_body,
        grid=(grid,),
        in_specs=[
            pl.BlockSpec((MUL, EDGE_BLK), lambda i: (0, i)),
            pl.BlockSpec((EDGE_BLK,), lambda i: (i,)),
            pl.BlockSpec((EDGE_BLK,), lambda i: (i,)),
        ],
        out_specs=pl.BlockSpec((EDGE_BLK, MUL), lambda i: (i, 0)),
        out_shape=jax.ShapeDtypeStruct((E_PAD, MUL), jnp.float32),
    )(valuesT, exh, ssrc)


def _sum_halves_body(p_ref, o_ref):
    o_ref[...] = (p_ref[0] + p_ref[1]) * (1.0 / NUM_NEIGHBORS)


def _sum_halves(out_p):
    return pl.pallas_call(
        _sum_halves_body,
        grid=(1,),
        in_specs=[pl.BlockSpec((2, _N_PAD, MUL), lambda i: (0, 0, 0))],
        out_specs=pl.BlockSpec((_N_PAD, MUL), lambda i: (0, 0)),
        out_shape=jax.ShapeDtypeStruct((_N_PAD, MUL), jnp.float32),
    )(out_p)


def _edge_compute_body(xs_ref, sq_ref, wqT_ref, wsimT_ref, w1kT_ref, w2kT_ref,
                       w1vT_ref, w2vT_ref, logits_ref, values_ref):
    xsT = xs_ref[...].T
    sq = sq_ref[...]

    pos_mask = sq > 0.0
    vlen = jnp.where(pos_mask, jnp.sqrt(jnp.where(pos_mask, sq, 1.0)), 0.0)
    x_safe = jnp.where(pos_mask, vlen, 1.0)

    nvec = (jnp.arange(NUM_BASIS, dtype=jnp.int32) + 1).astype(jnp.float32)[:, None]
    radT = jnp.sin(nvec * (jnp.pi / MAX_RADIUS) * x_safe[None, :]) / x_safe
    bmask = jnp.logical_and(pos_mask, vlen < MAX_RADIUS)
    scale = (2.0 / MAX_RADIUS) ** 0.5 * (NUM_BASIS ** 0.5)
    radT = jnp.where(bmask, radT * scale, 0.0)

    y = 10.0 * (1.0 - vlen / MAX_RADIUS)
    ymask = y > 0.0
    cutoff = jnp.where(ymask, jnp.exp(-1.0 / jnp.where(ymask, y, 1.0)), 0.0)

    inv_sqrt_b = 1.0 / (NUM_BASIS ** 0.5)
    inv_sqrt_h = 1.0 / (HIDDEN ** 0.5)
    inv_sqrt_m = 1.0 / (MUL ** 0.5)

    def radial_t(w1T, w2T):
        h = jnp.dot(w1T, radT, preferred_element_type=jnp.float32) * inv_sqrt_b
        h = SILU_NORM * (h * jax.nn.sigmoid(h))
        return jnp.dot(w2T, h, preferred_element_type=jnp.float32) * inv_sqrt_h

    wkT = radial_t(w1kT_ref[...], w2kT_ref[...])
    wvT = radial_t(w1vT_ref[...], w2vT_ref[...])

    b = xsT.shape[1]
    xs_b = xsT[:, None, :]
    keyT = jnp.sum(wkT.reshape(MUL, MUL, b) * xs_b, axis=0) * inv_sqrt_m
    valT = jnp.sum(wvT.reshape(MUL, MUL, b) * xs_b, axis=0) * inv_sqrt_m

    qT = jnp.dot(wqT_ref[...], xsT, preferred_element_type=jnp.float32) * inv_sqrt_m
    tT = jnp.dot(wsimT_ref[...], qT, preferred_element_type=jnp.float32)
    sim = jnp.sum(tT * keyT, axis=0) * (1.0 / MUL)

    logits_ref[...] = cutoff * sim
    values_ref[...] = valT


def _edge_compute(x_src, sq, wq, wsim2d, w1k, w2k, w1v, w2v):
    grid = E_PAD // EDGE_BLK
    rep = lambda shape: pl.BlockSpec(shape, lambda i: tuple(0 for _ in shape))
    return pl.pallas_call(
        _edge_compute_body,
        grid=(grid,),
        in_specs=[
            pl.BlockSpec((EDGE_BLK, MUL), lambda i: (i, 0)),
            pl.BlockSpec((EDGE_BLK,), lambda i: (i,)),
            rep((MUL, MUL)),
            rep((MUL, MUL)),
            rep((HIDDEN, NUM_BASIS)),
            rep((MUL * MUL, HIDDEN)),
            rep((HIDDEN, NUM_BASIS)),
            rep((MUL * MUL, HIDDEN)),
        ],
        out_specs=[
            pl.BlockSpec((EDGE_BLK,), lambda i: (i,)),
            pl.BlockSpec((MUL, EDGE_BLK), lambda i: (0, i)),
        ],
        out_shape=[
            jax.ShapeDtypeStruct((E_PAD,), jnp.float32),
            jax.ShapeDtypeStruct((MUL, E_PAD), jnp.float32),
        ],
    )(x_src, sq, wq.T, wsim2d.T, w1k.T, w2k.T, w1v.T, w2v.T)


def kernel(x, pos, edge_index, W_query, W_sim, W1k, W2k, W1v, W2v):
    src = edge_index[0]
    dst = edge_index[1]

    x_src, sq1 = _sc_gather(x, pos.reshape(-1), src, dst)

    logits, valuesT = _edge_compute(
        x_src, sq1, W_query, W_sim[:, :, 0], W1k, W2k, W1v, W2v)

    (m_p,) = _sc_segmax(src, logits)
    exh, s_p = _sc_segsum(src, logits, m_p)
    (ssrc,) = _sc_gather_s(src, s_p)
    scaled = _scale_values(valuesT, exh, ssrc)
    zeros = jnp.zeros((_SL, MUL), jnp.float32)
    (out_p,) = _sc_scatter_out(dst, scaled, zeros)
    return _sum_halves(out_p)[:N_NODES]


# speedup vs baseline: 1.0235x; 1.0235x over previous
import functools

import jax
import jax.numpy as jnp
from jax import lax
from jax.experimental import pallas as pl
from jax.experimental.pallas import tpu as pltpu
from jax.experimental.pallas import tpu_sc as plsc

N_NODES = 10000
N_EDGES = 160000
MUL = 16
NUM_BASIS = 32
MAX_RADIUS = 2.5
NUM_NEIGHBORS = 16
HIDDEN = 32
SILU_NORM = 1.6790

E_PAD = 163840
EDGE_BLK = 2048

_NW = 32
_BE = 128
_NB = N_EDGES // _BE
_MAXB = (_NB + _NW - 1) // _NW

_SC_MESH = plsc.VectorSubcoreMesh(core_axis_name="c", subcore_axis_name="s")


_U = 4
_FULL = (_NB // _NW) // _U


def _sc_gather_body(x_hbm, pos_hbm, src_hbm, dst_hbm, xsrc_out, sq_out,
                    pos_v, idx_s, idx_d, rows, sq_v, semi, semg, semo):
    w = lax.axis_index("s") * 2 + lax.axis_index("c")
    pltpu.sync_copy(pos_hbm, pos_v)

    def compute_sq(u, idx_s_u, idx_d_u, sq_u):
        for i in range(_BE // 16):
            si = idx_s_u[pl.ds(i * 16, 16)] * 3
            di = idx_d_u[pl.ds(i * 16, 16)] * 3
            dx = plsc.load_gather(pos_v, [si]) - plsc.load_gather(pos_v, [di])
            dy = plsc.load_gather(pos_v, [si + 1]) - plsc.load_gather(pos_v, [di + 1])
            dz = plsc.load_gather(pos_v, [si + 2]) - plsc.load_gather(pos_v, [di + 2])
            sq_u[pl.ds(i * 16, 16)] = dx * dx + dy * dy + dz * dz

    def body(j, carry):
        bases = [(w + _NW * (_U * j + u)) * _BE for u in range(_U)]
        cps = []
        for u, base in enumerate(bases):
            cps.append(pltpu.async_copy(src_hbm.at[pl.ds(base, _BE)], idx_s.at[u], semi))
            cps.append(pltpu.async_copy(dst_hbm.at[pl.ds(base, _BE)], idx_d.at[u], semi))
        for cp in cps:
            cp.wait()
        gs = [pltpu.async_copy(x_hbm.at[idx_s.at[u]], rows.at[u], semg)
              for u in range(_U)]
        for u in range(_U):
            compute_sq(u, idx_s.at[u], idx_d.at[u], sq_v.at[u])
        os = []
        for u, base in enumerate(bases):
            gs[u].wait()
            os.append(pltpu.async_copy(rows.at[u], xsrc_out.at[pl.ds(base, _BE)], semo))
            os.append(pltpu.async_copy(sq_v.at[u], sq_out.at[pl.ds(base, _BE)], semo))
        for cp in os:
            cp.wait()
        return carry

    lax.fori_loop(0, _FULL, body, 0)

    def tail(j, carry):
        b = w + _NW * j

        @pl.when(b < _NB)
        def _():
            base = b * _BE
            pltpu.sync_copy(src_hbm.at[pl.ds(base, _BE)], idx_s.at[0])
            pltpu.sync_copy(dst_hbm.at[pl.ds(base, _BE)], idx_d.at[0])
            cp = pltpu.async_copy(x_hbm.at[idx_s.at[0]], rows.at[0], semg)
            compute_sq(0, idx_s.at[0], idx_d.at[0], sq_v.at[0])
            cp.wait()
            pltpu.sync_copy(rows.at[0], xsrc_out.at[pl.ds(base, _BE)])
            pltpu.sync_copy(sq_v.at[0], sq_out.at[pl.ds(base, _BE)])

        return carry

    lax.fori_loop(_FULL * _U, _MAXB, tail, 0)


@functools.partial(
    pl.kernel,
    mesh=_SC_MESH,
    compiler_params=pltpu.CompilerParams(
        needs_layout_passes=False, use_tc_tiling_on_sc=False),
    out_type=[
        jax.ShapeDtypeStruct((E_PAD, MUL), jnp.float32),
        jax.ShapeDtypeStruct((E_PAD,), jnp.float32),
    ],
    scratch_types=[
        pltpu.VMEM((N_NODES * 3,), jnp.float32),
        pltpu.VMEM((_U, _BE), jnp.int32),
        pltpu.VMEM((_U, _BE), jnp.int32),
        pltpu.VMEM((_U, _BE, MUL), jnp.float32),
        pltpu.VMEM((_U, _BE), jnp.float32),
        pltpu.SemaphoreType.DMA,
        pltpu.SemaphoreType.DMA,
        pltpu.SemaphoreType.DMA,
    ],
)
def _sc_gather(x_hbm, pos_hbm, src_hbm, dst_hbm, xsrc_out, sq_out,
               pos_v, idx_s, idx_d, rows, sq_v, semi, semg, semo):
    _sc_gather_body(x_hbm, pos_hbm, src_hbm, dst_hbm, xsrc_out, sq_out,
                    pos_v, idx_s, idx_d, rows, sq_v, semi, semg, semo)


_N_PAD = 10240
_SL = _N_PAD // 16
_SC_PARAMS = pltpu.CompilerParams(
    needs_layout_passes=False, use_tc_tiling_on_sc=False)
_IOTA16 = None


def _iota16():
    return jnp.arange(16, dtype=jnp.int32)


def _worker_id():
    return lax.axis_index("s") * 2 + lax.axis_index("c")


def _combine_dups(sk, sv, kb, vb, is_max):
    iota = _iota16()
    kb[...] = sk
    vb[...] = sv
    for d in (1, 2, 4, 8):
        g = jnp.maximum(iota - d, 0)
        ks = plsc.load_gather(kb, [g])
        vs = plsc.load_gather(vb, [g])
        comb = jnp.maximum(sv, vs) if is_max else sv + vs
        sv = jnp.where(jnp.logical_and(ks == sk, iota >= d), comb, sv)
        vb[...] = sv
    kn = plsc.load_gather(kb, [jnp.minimum(iota + 1, 15)])
    is_last = jnp.logical_or(sk != kn, iota == 15)
    return sv, is_last


def _spmem_combine(acc_v, sh, part_hbm, buf, is_max):
    sid = lax.axis_index("s")
    cid = lax.axis_index("c")
    pltpu.sync_copy(acc_v, sh.at[sid])
    plsc.subcore_barrier()
    pltpu.sync_copy(sh.at[:, pl.ds(sid * _SL, _SL)], buf)

    def red(j, c):
        o = j * 16
        v = buf[0, pl.ds(o, 16)]
        for k in range(1, 16):
            vk = buf[k, pl.ds(o, 16)]
            v = jnp.maximum(v, vk) if is_max else v + vk
        acc_v[pl.ds(o, 16)] = v
        return c

    lax.fori_loop(0, _SL // 16, red, 0)
    pltpu.sync_copy(acc_v.at[pl.ds(0, _SL)], part_hbm.at[cid, pl.ds(sid * _SL, _SL)])


@functools.partial(
    pl.kernel,
    mesh=_SC_MESH,
    compiler_params=_SC_PARAMS,
    out_type=[jax.ShapeDtypeStruct((2, _N_PAD), jnp.float32)],
    scratch_types=[
        pltpu.VMEM((_N_PAD,), jnp.float32),
        pltpu.VMEM((_U, _BE), jnp.int32),
        pltpu.VMEM((_U, _BE), jnp.float32),
        pltpu.VMEM((16,), jnp.int32),
        pltpu.VMEM((16,), jnp.float32),
        pltpu.VMEM((16, _SL), jnp.float32),
        pltpu.VMEM_SHARED((16, _N_PAD), jnp.float32),
        pltpu.SemaphoreType.DMA,
    ],
)
def _sc_segmax(src_hbm, logits_hbm, m_p, m_t, idx_v, val_v, kb, vb, buf, sh, semi):
    w = _worker_id()
    neg = jnp.full((16,), -3.0e38, jnp.float32)

    def initb(i, c):
        m_t[pl.ds(i * 16, 16)] = neg
        return c

    lax.fori_loop(0, _N_PAD // 16, initb, 0)

    def blk(u):
        for i in range(_BE // 16):
            k = idx_v.at[u][pl.ds(i * 16, 16)]
            v = val_v.at[u][pl.ds(i * 16, 16)]
            sk, sv = plsc.sort_key_val(k, v)
            sv, is_last = _combine_dups(sk, sv, kb, vb, True)
            cur = plsc.load_gather(m_t, [sk])
            plsc.store_scatter(m_t, [sk], jnp.maximum(cur, sv), mask=is_last)

    def body(j, c):
        bases = [(w + _NW * (_U * j + u)) * _BE for u in range(_U)]
        cps = []
        for u, base in enumerate(bases):
            cps.append(pltpu.async_copy(src_hbm.at[pl.ds(base, _BE)], idx_v.at[u], semi))
            cps.append(pltpu.async_copy(logits_hbm.at[pl.ds(base, _BE)], val_v.at[u], semi))
        for cp in cps:
            cp.wait()
        for u in range(_U):
            blk(u)
        return c

    lax.fori_loop(0, _FULL, body, 0)

    def tail(j, c):
        b = w + _NW * j

        @pl.when(b < _NB)
        def _():
            base = b * _BE
            pltpu.sync_copy(src_hbm.at[pl.ds(base, _BE)], idx_v.at[0])
            pltpu.sync_copy(logits_hbm.at[pl.ds(base, _BE)], val_v.at[0])
            blk(0)

        return c

    lax.fori_loop(_FULL * _U, _MAXB, tail, 0)
    _spmem_combine(m_t, sh, m_p, buf, True)


@functools.partial(
    pl.kernel,
    mesh=_SC_MESH,
    compiler_params=_SC_PARAMS,
    out_type=[
        jax.ShapeDtypeStruct((E_PAD,), jnp.float32),
        jax.ShapeDtypeStruct((2, _N_PAD), jnp.float32),
    ],
    scratch_types=[
        pltpu.VMEM((_N_PAD,), jnp.float32),
        pltpu.VMEM((_N_PAD,), jnp.float32),
        pltpu.VMEM((_N_PAD,), jnp.float32),
        pltpu.VMEM((_U, _BE), jnp.int32),
        pltpu.VMEM((_U, _BE), jnp.float32),
        pltpu.VMEM((_U, _BE), jnp.float32),
        pltpu.VMEM((16,), jnp.int32),
        pltpu.VMEM((16,), jnp.float32),
        pltpu.VMEM((16, _SL), jnp.float32),
        pltpu.VMEM_SHARED((16, _N_PAD), jnp.float32),
        pltpu.SemaphoreType.DMA,
        pltpu.SemaphoreType.DMA,
    ],
)
def _sc_segsum(src_hbm, logits_hbm, m_p_hbm, exh_out, s_p,
               ma, mb, s_t, idx_v, val_v, eh_v, kb, vb, buf, sh, semi, semo):
    w = _worker_id()
    pltpu.sync_copy(m_p_hbm.at[0], ma)
    pltpu.sync_copy(m_p_hbm.at[1], mb)

    def mmax(i, c):
        o = i * 16
        ma[pl.ds(o, 16)] = jnp.maximum(ma[pl.ds(o, 16)], mb[pl.ds(o, 16)])
        return c

    lax.fori_loop(0, _N_PAD // 16, mmax, 0)

    zv = jnp.zeros((16,), jnp.float32)

    def initb(i, c):
        s_t[pl.ds(i * 16, 16)] = zv
        return c

    lax.fori_loop(0, _N_PAD // 16, initb, 0)

    def blk(u):
        for i in range(_BE // 16):
            k = idx_v.at[u][pl.ds(i * 16, 16)]
            lg = val_v.at[u][pl.ds(i * 16, 16)]
            mg = plsc.load_gather(ma, [k])
            eh = jnp.exp(0.5 * (lg - mg))
            eh_v.at[u][pl.ds(i * 16, 16)] = eh
            sk, sv = plsc.sort_key_val(k, eh * eh)
            sv, is_last = _combine_dups(sk, sv, kb, vb, False)
            cur = plsc.load_gather(s_t, [sk])
            plsc.store_scatter(s_t, [sk], cur + sv, mask=is_last)

    def body(j, c):
        bases = [(w + _NW * (_U * j + u)) * _BE for u in range(_U)]
        cps = []
        for u, base in enumerate(bases):
            cps.append(pltpu.async_copy(src_hbm.at[pl.ds(base, _BE)], idx_v.at[u], semi))
            cps.append(pltpu.async_copy(logits_hbm.at[pl.ds(base, _BE)], val_v.at[u], semi))
        for cp in cps:
            cp.wait()
        os = []
        for u, base in enumerate(bases):
            blk(u)
            os.append(pltpu.async_copy(eh_v.at[u], exh_out.at[pl.ds(base, _BE)], semo))
        for cp in os:
            cp.wait()
        return c

    lax.fori_loop(0, _FULL, body, 0)

    def tail(j, c):
        b = w + _NW * j

        @pl.when(b < _NB)
        def _():
            base = b * _BE
            pltpu.sync_copy(src_hbm.at[pl.ds(base, _BE)], idx_v.at[0])
            pltpu.sync_copy(logits_hbm.at[pl.ds(base, _BE)], val_v.at[0])
            blk(0)
            pltpu.sync_copy(eh_v.at[0], exh_out.at[pl.ds(base, _BE)])

        return c

    lax.fori_loop(_FULL * _U, _MAXB, tail, 0)
    _spmem_combine(s_t, sh, s_p, buf, False)


@functools.partial(
    pl.kernel,
    mesh=_SC_MESH,
    compiler_params=_SC_PARAMS,
    out_type=[jax.ShapeDtypeStruct((E_PAD,), jnp.float32)],
    scratch_types=[
        pltpu.VMEM((_N_PAD,), jnp.float32),
        pltpu.VMEM((_N_PAD,), jnp.float32),
        pltpu.VMEM((_U, _BE), jnp.int32),
        pltpu.VMEM((_U, _BE), jnp.float32),
        pltpu.SemaphoreType.DMA,
        pltpu.SemaphoreType.DMA,
    ],
)
def _sc_gather_s(src_hbm, s_p_hbm, ssrc_out, sa, sb, idx_v, sg_v, semi, semo):
    w = _worker_id()
    pltpu.sync_copy(s_p_hbm.at[0], sa)
    pltpu.sync_copy(s_p_hbm.at[1], sb)

    def madd(i, c):
        o = i * 16
        sa[pl.ds(o, 16)] = sa[pl.ds(o, 16)] + sb[pl.ds(o, 16)]
        return c

    lax.fori_loop(0, _N_PAD // 16, madd, 0)

    def blk(u):
        for i in range(_BE // 16):
            k = idx_v.at[u][pl.ds(i * 16, 16)]
            sg_v.at[u][pl.ds(i * 16, 16)] = plsc.load_gather(sa, [k])

    def body(j, c):
        bases = [(w + _NW * (_U * j + u)) * _BE for u in range(_U)]
        cps = [pltpu.async_copy(src_hbm.at[pl.ds(base, _BE)], idx_v.at[u], semi)
               for u, base in enumerate(bases)]
        for cp in cps:
            cp.wait()
        os = []
        for u, base in enumerate(bases):
            blk(u)
            os.append(pltpu.async_copy(sg_v.at[u], ssrc_out.at[pl.ds(base, _BE)], semo))
        for cp in os:
            cp.wait()
        return c

    lax.fori_loop(0, _FULL, body, 0)

    def tail(j, c):
        b = w + _NW * j

        @pl.when(b < _NB)
        def _():
            base = b * _BE
            pltpu.sync_copy(src_hbm.at[pl.ds(base, _BE)], idx_v.at[0])
            blk(0)
            pltpu.sync_copy(sg_v.at[0], ssrc_out.at[pl.ds(base, _BE)])

        return c

    lax.fori_loop(_FULL * _U, _MAXB, tail, 0)


@functools.partial(
    pl.kernel,
    mesh=_SC_MESH,
    compiler_params=_SC_PARAMS,
    out_type=[jax.ShapeDtypeStruct((2, _N_PAD, MUL), jnp.float32)],
    scratch_types=[
        pltpu.VMEM((_U, _BE), jnp.int32),
        pltpu.VMEM((_U, _BE, MUL), jnp.float32),
        pltpu.VMEM_SHARED((_N_PAD, MUL), jnp.float32),
        pltpu.SemaphoreType.DMA,
    ],
)
def _sc_scatter_out(dst_hbm, scaled_hbm, zeros_hbm, out_p, idx_v, rows_v, oacc,
                    semi):
    w = _worker_id()
    sid = lax.axis_index("s")
    cid = lax.axis_index("c")
    pltpu.sync_copy(zeros_hbm, oacc.at[pl.ds(sid * _SL, _SL)])
    plsc.subcore_barrier()

    def body(j, c):
        bases = [(w + _NW * (_U * j + u)) * _BE for u in range(_U)]
        cps = []
        for u, base in enumerate(bases):
            cps.append(pltpu.async_copy(dst_hbm.at[pl.ds(base, _BE)], idx_v.at[u], semi))
            cps.append(pltpu.async_copy(scaled_hbm.at[pl.ds(base, _BE)], rows_v.at[u], semi))
        for cp in cps:
            cp.wait()
        for u in range(_U):
            pltpu.sync_copy(rows_v.at[u], oacc.at[idx_v.at[u]], add=True)
        return c

    lax.fori_loop(0, _FULL, body, 0)

    def tail(j, c):
        b = w + _NW * j

        @pl.when(b < _NB)
        def _():
            base = b * _BE
            pltpu.sync_copy(dst_hbm.at[pl.ds(base, _BE)], idx_v.at[0])
            pltpu.sync_copy(scaled_hbm.at[pl.ds(base, _BE)], rows_v.at[0])
            pltpu.sync_copy(rows_v.at[0], oacc.at[idx_v.at[0]], add=True)

        return c

    lax.fori_loop(_FULL * _U, _MAXB, tail, 0)
    plsc.subcore_barrier()
    pltpu.sync_copy(oacc.at[pl.ds(sid * _SL, _SL)],
                    out_p.at[cid, pl.ds(sid * _SL, _SL)])


def _scale_body(vT_ref, eh_ref, ss_ref, o_ref):
    wgt = eh_ref[...] * jax.lax.rsqrt(ss_ref[...])
    o_ref[...] = (vT_ref[...] * wgt).T


def _scale_values(valuesT, exh, ssrc):
    grid = E_PAD // EDGE_BLK
    return pl.pallas_call(
        _scale_body,
        grid=(grid,),
        in_specs=[
            pl.BlockSpec((MUL, EDGE_BLK), lambda i: (0, i)),
            pl.BlockSpec((EDGE_BLK,), lambda i: (i,)),
            pl.BlockSpec((EDGE_BLK,), lambda i: (i,)),
        ],
        out_specs=pl.BlockSpec((EDGE_BLK, MUL), lambda i: (i, 0)),
        out_shape=jax.ShapeDtypeStruct((E_PAD, MUL), jnp.float32),
    )(valuesT, exh, ssrc)


def _sum_halves_body(p_ref, o_ref):
    o_ref[...] = (p_ref[0] + p_ref[1]) * (1.0 / NUM_NEIGHBORS)


def _sum_halves(out_p):
    return pl.pallas_call(
        _sum_halves_body,
        grid=(1,),
        in_specs=[pl.BlockSpec((2, _N_PAD, MUL), lambda i: (0, 0, 0))],
        out_specs=pl.BlockSpec((_N_PAD, MUL), lambda i: (0, 0)),
        out_shape=jax.ShapeDtypeStruct((_N_PAD, MUL), jnp.float32),
    )(out_p)


def _edge_compute_body(xs_ref, sq_ref, wqT_ref, wsimT_ref, w1kT_ref, w2kT_ref,
                       w1vT_ref, w2vT_ref, logits_ref, values_ref):
    xsT = xs_ref[...].T
    sq = sq_ref[...]

    pos_mask = sq > 0.0
    vlen = jnp.where(pos_mask, jnp.sqrt(jnp.where(pos_mask, sq, 1.0)), 0.0)
    x_safe = jnp.where(pos_mask, vlen, 1.0)

    nvec = (jnp.arange(NUM_BASIS, dtype=jnp.int32) + 1).astype(jnp.float32)[:, None]
    radT = jnp.sin(nvec * (jnp.pi / MAX_RADIUS) * x_safe[None, :]) / x_safe
    bmask = jnp.logical_and(pos_mask, vlen < MAX_RADIUS)
    scale = (2.0 / MAX_RADIUS) ** 0.5 * (NUM_BASIS ** 0.5)
    radT = jnp.where(bmask, radT * scale, 0.0)

    y = 10.0 * (1.0 - vlen / MAX_RADIUS)
    ymask = y > 0.0
    cutoff = jnp.where(ymask, jnp.exp(-1.0 / jnp.where(ymask, y, 1.0)), 0.0)

    inv_sqrt_b = 1.0 / (NUM_BASIS ** 0.5)
    inv_sqrt_h = 1.0 / (HIDDEN ** 0.5)
    inv_sqrt_m = 1.0 / (MUL ** 0.5)

    def radial_t(w1T, w2T):
        h = jnp.dot(w1T, radT, preferred_element_type=jnp.float32) * inv_sqrt_b
        h = SILU_NORM * (h * jax.nn.sigmoid(h))
        return jnp.dot(w2T, h, preferred_element_type=jnp.float32) * inv_sqrt_h

    wkT = radial_t(w1kT_ref[...], w2kT_ref[...])
    wvT = radial_t(w1vT_ref[...], w2vT_ref[...])

    b = xsT.shape[1]
    xs_b = xsT[:, None, :]
    keyT = jnp.sum(wkT.reshape(MUL, MUL, b) * xs_b, axis=0) * inv_sqrt_m
    valT = jnp.sum(wvT.reshape(MUL, MUL, b) * xs_b, axis=0) * inv_sqrt_m

    qT = jnp.dot(wqT_ref[...], xsT, preferred_element_type=jnp.float32) * inv_sqrt_m
    tT = jnp.dot(wsimT_ref[...], qT, preferred_element_type=jnp.float32)
    sim = jnp.sum(tT * keyT, axis=0) * (1.0 / MUL)

    logits_ref[...] = cutoff * sim
    values_ref[...] = valT


def _edge_compute(x_src, sq, wq, wsim2d, w1k, w2k, w1v, w2v):
    grid = E_PAD // EDGE_BLK
    rep = lambda shape: pl.BlockSpec(shape, lambda i: tuple(0 for _ in shape))
    return pl.pallas_call(
        _edge_compute_body,
        grid=(grid,),
        in_specs=[
            pl.BlockSpec((EDGE_BLK, MUL), lambda i: (i, 0)),
            pl.BlockSpec((EDGE_BLK,), lambda i: (i,)),
            rep((MUL, MUL)),
            rep((MUL, MUL)),
            rep((HIDDEN, NUM_BASIS)),
            rep((MUL * MUL, HIDDEN)),
            rep((HIDDEN, NUM_BASIS)),
            rep((MUL * MUL, HIDDEN)),
        ],
        out_specs=[
            pl.BlockSpec((EDGE_BLK,), lambda i: (i,)),
            pl.BlockSpec((MUL, EDGE_BLK), lambda i: (0, i)),
        ],
        out_shape=[
            jax.ShapeDtypeStruct((E_PAD,), jnp.float32),
            jax.ShapeDtypeStruct((MUL, E_PAD), jnp.float32),
        ],
    )(x_src, sq, wq.T, wsim2d.T, w1k.T, w2k.T, w1v.T, w2v.T)


def kernel(x, pos, edge_index, W_query, W_sim, W1k, W2k, W1v, W2v):
    src = edge_index[0]
    dst = edge_index[1]

    x_src, sq1 = _sc_gather(x, pos.reshape(-1), src, dst)

    logits, valuesT = _edge_compute(
        x_src, sq1, W_query, W_sim[:, :, 0], W1k, W2k, W1v, W2v)

    (m_p,) = _sc_segmax(src, logits)
    exh, s_p = _sc_segsum(src, logits, m_p)
    (ssrc,) = _sc_gather_s(src, s_p)
    scaled = _scale_values(valuesT, exh, ssrc)
    zeros = jnp.zeros((_SL, MUL), jnp.float32)
    (out_p,) = _sc_scatter_out(dst, scaled, zeros)
    return _sum_halves(out_p)[:N_NODES]


# speedup vs baseline: 1.2193x; 1.1913x over previous
import functools

import jax
import jax.numpy as jnp
from jax import lax
from jax.experimental import pallas as pl
from jax.experimental.pallas import tpu as pltpu
from jax.experimental.pallas import tpu_sc as plsc

N_NODES = 10000
N_EDGES = 160000
MUL = 16
NUM_BASIS = 32
MAX_RADIUS = 2.5
NUM_NEIGHBORS = 16
HIDDEN = 32
SILU_NORM = 1.6790

E_PAD = 163840
EDGE_BLK = 2048

_NW = 32
_BE = 128
_NB = N_EDGES // _BE
_MAXB = (_NB + _NW - 1) // _NW

_SC_MESH = plsc.VectorSubcoreMesh(core_axis_name="c", subcore_axis_name="s")


_U = 4
_FULL = (_NB // _NW) // _U


def _sc_gather_body(x_hbm, pos_hbm, src_hbm, dst_hbm, xsrc_out, sq_out,
                    pos_v, idx_s, idx_d, rows, sq_v, semi, semg, semo):
    w = lax.axis_index("s") * 2 + lax.axis_index("c")
    pltpu.sync_copy(pos_hbm, pos_v)

    def compute_sq(u, idx_s_u, idx_d_u, sq_u):
        for i in range(_BE // 16):
            si = idx_s_u[pl.ds(i * 16, 16)] * 3
            di = idx_d_u[pl.ds(i * 16, 16)] * 3
            dx = plsc.load_gather(pos_v, [si]) - plsc.load_gather(pos_v, [di])
            dy = plsc.load_gather(pos_v, [si + 1]) - plsc.load_gather(pos_v, [di + 1])
            dz = plsc.load_gather(pos_v, [si + 2]) - plsc.load_gather(pos_v, [di + 2])
            sq_u[pl.ds(i * 16, 16)] = dx * dx + dy * dy + dz * dz

    def body(j, carry):
        bases = [(w + _NW * (_U * j + u)) * _BE for u in range(_U)]
        cps = []
        for u, base in enumerate(bases):
            cps.append(pltpu.async_copy(src_hbm.at[pl.ds(base, _BE)], idx_s.at[u], semi))
            cps.append(pltpu.async_copy(dst_hbm.at[pl.ds(base, _BE)], idx_d.at[u], semi))
        for cp in cps:
            cp.wait()
        gs = [pltpu.async_copy(x_hbm.at[idx_s.at[u]], rows.at[u], semg)
              for u in range(_U)]
        for u in range(_U):
            compute_sq(u, idx_s.at[u], idx_d.at[u], sq_v.at[u])
        os = []
        for u, base in enumerate(bases):
            gs[u].wait()
            os.append(pltpu.async_copy(rows.at[u], xsrc_out.at[pl.ds(base, _BE)], semo))
            os.append(pltpu.async_copy(sq_v.at[u], sq_out.at[pl.ds(base, _BE)], semo))
        for cp in os:
            cp.wait()
        return carry

    lax.fori_loop(0, _FULL, body, 0)

    def tail(j, carry):
        b = w + _NW * j

        @pl.when(b < _NB)
        def _():
            base = b * _BE
            pltpu.sync_copy(src_hbm.at[pl.ds(base, _BE)], idx_s.at[0])
            pltpu.sync_copy(dst_hbm.at[pl.ds(base, _BE)], idx_d.at[0])
            cp = pltpu.async_copy(x_hbm.at[idx_s.at[0]], rows.at[0], semg)
            compute_sq(0, idx_s.at[0], idx_d.at[0], sq_v.at[0])
            cp.wait()
            pltpu.sync_copy(rows.at[0], xsrc_out.at[pl.ds(base, _BE)])
            pltpu.sync_copy(sq_v.at[0], sq_out.at[pl.ds(base, _BE)])

        return carry

    lax.fori_loop(_FULL * _U, _MAXB, tail, 0)


@functools.partial(
    pl.kernel,
    mesh=_SC_MESH,
    compiler_params=pltpu.CompilerParams(
        needs_layout_passes=False, use_tc_tiling_on_sc=False),
    out_type=[
        jax.ShapeDtypeStruct((E_PAD, MUL), jnp.float32),
        jax.ShapeDtypeStruct((E_PAD,), jnp.float32),
    ],
    scratch_types=[
        pltpu.VMEM((N_NODES * 3,), jnp.float32),
        pltpu.VMEM((_U, _BE), jnp.int32),
        pltpu.VMEM((_U, _BE), jnp.int32),
        pltpu.VMEM((_U, _BE, MUL), jnp.float32),
        pltpu.VMEM((_U, _BE), jnp.float32),
        pltpu.SemaphoreType.DMA,
        pltpu.SemaphoreType.DMA,
        pltpu.SemaphoreType.DMA,
    ],
)
def _sc_gather(x_hbm, pos_hbm, src_hbm, dst_hbm, xsrc_out, sq_out,
               pos_v, idx_s, idx_d, rows, sq_v, semi, semg, semo):
    _sc_gather_body(x_hbm, pos_hbm, src_hbm, dst_hbm, xsrc_out, sq_out,
                    pos_v, idx_s, idx_d, rows, sq_v, semi, semg, semo)


_N_PAD = 10240
_SL = _N_PAD // 16
_SC_PARAMS = pltpu.CompilerParams(
    needs_layout_passes=False, use_tc_tiling_on_sc=False)
_IOTA16 = None


def _iota16():
    return jnp.arange(16, dtype=jnp.int32)


def _worker_id():
    return lax.axis_index("s") * 2 + lax.axis_index("c")


def _combine_dups(sk, sv, kb, vb, is_max):
    iota = _iota16()
    kb[...] = sk
    vb[...] = sv
    for d in (1, 2, 4, 8):
        g = jnp.maximum(iota - d, 0)
        ks = plsc.load_gather(kb, [g])
        vs = plsc.load_gather(vb, [g])
        comb = jnp.maximum(sv, vs) if is_max else sv + vs
        sv = jnp.where(jnp.logical_and(ks == sk, iota >= d), comb, sv)
        vb[...] = sv
    kn = plsc.load_gather(kb, [jnp.minimum(iota + 1, 15)])
    is_last = jnp.logical_or(sk != kn, iota == 15)
    return sv, is_last


def _spmem_combine(acc_v, sh, part_hbm, buf, is_max):
    sid = lax.axis_index("s")
    cid = lax.axis_index("c")
    pltpu.sync_copy(acc_v, sh.at[sid])
    plsc.subcore_barrier()
    pltpu.sync_copy(sh.at[:, pl.ds(sid * _SL, _SL)], buf)

    def red(j, c):
        o = j * 16
        v = buf[0, pl.ds(o, 16)]
        for k in range(1, 16):
            vk = buf[k, pl.ds(o, 16)]
            v = jnp.maximum(v, vk) if is_max else v + vk
        acc_v[pl.ds(o, 16)] = v
        return c

    lax.fori_loop(0, _SL // 16, red, 0)
    pltpu.sync_copy(acc_v.at[pl.ds(0, _SL)], part_hbm.at[cid, pl.ds(sid * _SL, _SL)])


@functools.partial(
    pl.kernel,
    mesh=_SC_MESH,
    compiler_params=_SC_PARAMS,
    out_type=[jax.ShapeDtypeStruct((2, _N_PAD), jnp.float32)],
    scratch_types=[
        pltpu.VMEM((_N_PAD,), jnp.float32),
        pltpu.VMEM((_U, _BE), jnp.int32),
        pltpu.VMEM((_U, _BE), jnp.float32),
        pltpu.VMEM((16,), jnp.int32),
        pltpu.VMEM((16,), jnp.float32),
        pltpu.VMEM((16, _SL), jnp.float32),
        pltpu.VMEM_SHARED((16, _N_PAD), jnp.float32),
        pltpu.SemaphoreType.DMA,
    ],
)
def _sc_segmax(src_hbm, logits_hbm, m_p, m_t, idx_v, val_v, kb, vb, buf, sh, semi):
    w = _worker_id()
    neg = jnp.full((16,), -3.0e38, jnp.float32)

    def initb(i, c):
        m_t[pl.ds(i * 16, 16)] = neg
        return c

    lax.fori_loop(0, _N_PAD // 16, initb, 0)

    def blk(u):
        for i in range(_BE // 16):
            k = idx_v.at[u][pl.ds(i * 16, 16)]
            v = val_v.at[u][pl.ds(i * 16, 16)]
            sk, sv = plsc.sort_key_val(k, v)
            sv, is_last = _combine_dups(sk, sv, kb, vb, True)
            cur = plsc.load_gather(m_t, [sk])
            plsc.store_scatter(m_t, [sk], jnp.maximum(cur, sv), mask=is_last)

    def body(j, c):
        bases = [(w + _NW * (_U * j + u)) * _BE for u in range(_U)]
        cps = []
        for u, base in enumerate(bases):
            cps.append(pltpu.async_copy(src_hbm.at[pl.ds(base, _BE)], idx_v.at[u], semi))
            cps.append(pltpu.async_copy(logits_hbm.at[pl.ds(base, _BE)], val_v.at[u], semi))
        for cp in cps:
            cp.wait()
        for u in range(_U):
            blk(u)
        return c

    lax.fori_loop(0, _FULL, body, 0)

    def tail(j, c):
        b = w + _NW * j

        @pl.when(b < _NB)
        def _():
            base = b * _BE
            pltpu.sync_copy(src_hbm.at[pl.ds(base, _BE)], idx_v.at[0])
            pltpu.sync_copy(logits_hbm.at[pl.ds(base, _BE)], val_v.at[0])
            blk(0)

        return c

    lax.fori_loop(_FULL * _U, _MAXB, tail, 0)
    _spmem_combine(m_t, sh, m_p, buf, True)


@functools.partial(
    pl.kernel,
    mesh=_SC_MESH,
    compiler_params=_SC_PARAMS,
    out_type=[
        jax.ShapeDtypeStruct((E_PAD,), jnp.float32),
        jax.ShapeDtypeStruct((2, _N_PAD), jnp.float32),
    ],
    scratch_types=[
        pltpu.VMEM((_N_PAD,), jnp.float32),
        pltpu.VMEM((_N_PAD,), jnp.float32),
        pltpu.VMEM((_N_PAD,), jnp.float32),
        pltpu.VMEM((_U, _BE), jnp.int32),
        pltpu.VMEM((_U, _BE), jnp.float32),
        pltpu.VMEM((_U, _BE), jnp.float32),
        pltpu.VMEM((16,), jnp.int32),
        pltpu.VMEM((16,), jnp.float32),
        pltpu.VMEM((16, _SL), jnp.float32),
        pltpu.VMEM_SHARED((16, _N_PAD), jnp.float32),
        pltpu.SemaphoreType.DMA,
        pltpu.SemaphoreType.DMA,
    ],
)
def _sc_segsum(src_hbm, logits_hbm, m_p_hbm, exh_out, s_p,
               ma, mb, s_t, idx_v, val_v, eh_v, kb, vb, buf, sh, semi, semo):
    w = _worker_id()
    pltpu.sync_copy(m_p_hbm.at[0], ma)
    pltpu.sync_copy(m_p_hbm.at[1], mb)

    def mmax(i, c):
        o = i * 16
        ma[pl.ds(o, 16)] = jnp.maximum(ma[pl.ds(o, 16)], mb[pl.ds(o, 16)])
        return c

    lax.fori_loop(0, _N_PAD // 16, mmax, 0)

    zv = jnp.zeros((16,), jnp.float32)

    def initb(i, c):
        s_t[pl.ds(i * 16, 16)] = zv
        return c

    lax.fori_loop(0, _N_PAD // 16, initb, 0)

    def blk(u):
        for i in range(_BE // 16):
            k = idx_v.at[u][pl.ds(i * 16, 16)]
            lg = val_v.at[u][pl.ds(i * 16, 16)]
            mg = plsc.load_gather(ma, [k])
            eh = jnp.exp(0.5 * (lg - mg))
            eh_v.at[u][pl.ds(i * 16, 16)] = eh
            sk, sv = plsc.sort_key_val(k, eh * eh)
            sv, is_last = _combine_dups(sk, sv, kb, vb, False)
            cur = plsc.load_gather(s_t, [sk])
            plsc.store_scatter(s_t, [sk], cur + sv, mask=is_last)

    def body(j, c):
        bases = [(w + _NW * (_U * j + u)) * _BE for u in range(_U)]
        cps = []
        for u, base in enumerate(bases):
            cps.append(pltpu.async_copy(src_hbm.at[pl.ds(base, _BE)], idx_v.at[u], semi))
            cps.append(pltpu.async_copy(logits_hbm.at[pl.ds(base, _BE)], val_v.at[u], semi))
        for cp in cps:
            cp.wait()
        os = []
        for u, base in enumerate(bases):
            blk(u)
            os.append(pltpu.async_copy(eh_v.at[u], exh_out.at[pl.ds(base, _BE)], semo))
        for cp in os:
            cp.wait()
        return c

    lax.fori_loop(0, _FULL, body, 0)

    def tail(j, c):
        b = w + _NW * j

        @pl.when(b < _NB)
        def _():
            base = b * _BE
            pltpu.sync_copy(src_hbm.at[pl.ds(base, _BE)], idx_v.at[0])
            pltpu.sync_copy(logits_hbm.at[pl.ds(base, _BE)], val_v.at[0])
            blk(0)
            pltpu.sync_copy(eh_v.at[0], exh_out.at[pl.ds(base, _BE)])

        return c

    lax.fori_loop(_FULL * _U, _MAXB, tail, 0)
    _spmem_combine(s_t, sh, s_p, buf, False)


@functools.partial(
    pl.kernel,
    mesh=_SC_MESH,
    compiler_params=_SC_PARAMS,
    out_type=[jax.ShapeDtypeStruct((2, _N_PAD, MUL), jnp.float32)],
    scratch_types=[
        pltpu.VMEM((_U, _BE), jnp.int32),
        pltpu.VMEM((_U, _BE), jnp.int32),
        pltpu.VMEM((_U, _BE), jnp.float32),
        pltpu.VMEM((_U, MUL, _BE), jnp.float32),
        pltpu.VMEM((_U, _BE, MUL), jnp.float32),
        pltpu.VMEM((_N_PAD,), jnp.float32),
        pltpu.VMEM_SHARED((_N_PAD, MUL), jnp.float32),
        pltpu.SemaphoreType.DMA,
    ],
)
def _sc_scatter_out(dst_hbm, src_hbm, valT_hbm, exh_hbm, rs_hbm, zeros_hbm,
                    out_p, idx_d, idx_s, eh_b, colb, rows_v, rs_v, oacc, semi):
    w = _worker_id()
    sid = lax.axis_index("s")
    cid = lax.axis_index("c")
    pltpu.sync_copy(rs_hbm, rs_v)
    pltpu.sync_copy(zeros_hbm, oacc.at[pl.ds(sid * _SL, _SL)])
    plsc.subcore_barrier()
    iota = _iota16()

    def blk(u):
        for i in range(_BE // 16):
            ks = idx_s.at[u][pl.ds(i * 16, 16)]
            wv = eh_b.at[u][pl.ds(i * 16, 16)] * plsc.load_gather(rs_v, [ks])
            e16 = iota + i * 16
            for c in range(MUL):
                vc = colb.at[u][c, pl.ds(i * 16, 16)] * wv
                plsc.store_scatter(rows_v.at[u],
                                   [e16, jnp.full((16,), c, jnp.int32)], vc)

    def issue_loads(u, base):
        return [
            pltpu.async_copy(dst_hbm.at[pl.ds(base, _BE)], idx_d.at[u], semi),
            pltpu.async_copy(src_hbm.at[pl.ds(base, _BE)], idx_s.at[u], semi),
            pltpu.async_copy(exh_hbm.at[pl.ds(base, _BE)], eh_b.at[u], semi),
            pltpu.async_copy(valT_hbm.at[:, pl.ds(base, _BE)], colb.at[u], semi),
        ]

    def body(j, c):
        bases = [(w + _NW * (_U * j + u)) * _BE for u in range(_U)]
        cps = []
        for u, base in enumerate(bases):
            cps.extend(issue_loads(u, base))
        for cp in cps:
            cp.wait()
        for u in range(_U):
            blk(u)
            pltpu.sync_copy(rows_v.at[u], oacc.at[idx_d.at[u]], add=True)
        return c

    lax.fori_loop(0, _FULL, body, 0)

    def tail(j, c):
        b = w + _NW * j

        @pl.when(b < _NB)
        def _():
            base = b * _BE
            for cp in issue_loads(0, base):
                cp.wait()
            blk(0)
            pltpu.sync_copy(rows_v.at[0], oacc.at[idx_d.at[0]], add=True)

        return c

    lax.fori_loop(_FULL * _U, _MAXB, tail, 0)
    plsc.subcore_barrier()
    pltpu.sync_copy(oacc.at[pl.ds(sid * _SL, _SL)],
                    out_p.at[cid, pl.ds(sid * _SL, _SL)])


def _rsqrt_body(p_ref, o_ref):
    s = p_ref[0, :] + p_ref[1, :]
    o_ref[...] = jnp.where(s > 0, jax.lax.rsqrt(jnp.where(s > 0, s, 1.0)), 0.0)


def _rsqrt_nodes(s_p):
    return pl.pallas_call(
        _rsqrt_body,
        grid=(1,),
        in_specs=[pl.BlockSpec((2, _N_PAD), lambda i: (0, 0))],
        out_specs=pl.BlockSpec((_N_PAD,), lambda i: (0,)),
        out_shape=jax.ShapeDtypeStruct((_N_PAD,), jnp.float32),
    )(s_p)


def _sum_halves_body(p_ref, o_ref):
    o_ref[...] = (p_ref[0] + p_ref[1]) * (1.0 / NUM_NEIGHBORS)


def _sum_halves(out_p):
    return pl.pallas_call(
        _sum_halves_body,
        grid=(1,),
        in_specs=[pl.BlockSpec((2, _N_PAD, MUL), lambda i: (0, 0, 0))],
        out_specs=pl.BlockSpec((_N_PAD, MUL), lambda i: (0, 0)),
        out_shape=jax.ShapeDtypeStruct((_N_PAD, MUL), jnp.float32),
    )(out_p)


def _edge_compute_body(xs_ref, sq_ref, wqT_ref, wsimT_ref, w1kT_ref, w2kT_ref,
                       w1vT_ref, w2vT_ref, logits_ref, values_ref):
    xsT = xs_ref[...].T
    sq = sq_ref[...]

    pos_mask = sq > 0.0
    vlen = jnp.where(pos_mask, jnp.sqrt(jnp.where(pos_mask, sq, 1.0)), 0.0)
    x_safe = jnp.where(pos_mask, vlen, 1.0)

    nvec = (jnp.arange(NUM_BASIS, dtype=jnp.int32) + 1).astype(jnp.float32)[:, None]
    radT = jnp.sin(nvec * (jnp.pi / MAX_RADIUS) * x_safe[None, :]) / x_safe
    bmask = jnp.logical_and(pos_mask, vlen < MAX_RADIUS)
    scale = (2.0 / MAX_RADIUS) ** 0.5 * (NUM_BASIS ** 0.5)
    radT = jnp.where(bmask, radT * scale, 0.0)

    y = 10.0 * (1.0 - vlen / MAX_RADIUS)
    ymask = y > 0.0
    cutoff = jnp.where(ymask, jnp.exp(-1.0 / jnp.where(ymask, y, 1.0)), 0.0)

    inv_sqrt_b = 1.0 / (NUM_BASIS ** 0.5)
    inv_sqrt_h = 1.0 / (HIDDEN ** 0.5)
    inv_sqrt_m = 1.0 / (MUL ** 0.5)

    def radial_t(w1T, w2T):
        h = jnp.dot(w1T, radT, preferred_element_type=jnp.float32) * inv_sqrt_b
        h = SILU_NORM * (h * jax.nn.sigmoid(h))
        return jnp.dot(w2T, h, preferred_element_type=jnp.float32) * inv_sqrt_h

    wkT = radial_t(w1kT_ref[...], w2kT_ref[...])
    wvT = radial_t(w1vT_ref[...], w2vT_ref[...])

    b = xsT.shape[1]
    xs_b = xsT[:, None, :]
    keyT = jnp.sum(wkT.reshape(MUL, MUL, b) * xs_b, axis=0) * inv_sqrt_m
    valT = jnp.sum(wvT.reshape(MUL, MUL, b) * xs_b, axis=0) * inv_sqrt_m

    qT = jnp.dot(wqT_ref[...], xsT, preferred_element_type=jnp.float32) * inv_sqrt_m
    tT = jnp.dot(wsimT_ref[...], qT, preferred_element_type=jnp.float32)
    sim = jnp.sum(tT * keyT, axis=0) * (1.0 / MUL)

    logits_ref[...] = cutoff * sim
    values_ref[...] = valT


def _edge_compute(x_src, sq, wq, wsim2d, w1k, w2k, w1v, w2v):
    grid = E_PAD // EDGE_BLK
    rep = lambda shape: pl.BlockSpec(shape, lambda i: tuple(0 for _ in shape))
    return pl.pallas_call(
        _edge_compute_body,
        grid=(grid,),
        in_specs=[
            pl.BlockSpec((EDGE_BLK, MUL), lambda i: (i, 0)),
            pl.BlockSpec((EDGE_BLK,), lambda i: (i,)),
            rep((MUL, MUL)),
            rep((MUL, MUL)),
            rep((HIDDEN, NUM_BASIS)),
            rep((MUL * MUL, HIDDEN)),
            rep((HIDDEN, NUM_BASIS)),
            rep((MUL * MUL, HIDDEN)),
        ],
        out_specs=[
            pl.BlockSpec((EDGE_BLK,), lambda i: (i,)),
            pl.BlockSpec((MUL, EDGE_BLK), lambda i: (0, i)),
        ],
        out_shape=[
            jax.ShapeDtypeStruct((E_PAD,), jnp.float32),
            jax.ShapeDtypeStruct((MUL, E_PAD), jnp.float32),
        ],
    )(x_src, sq, wq.T, wsim2d.T, w1k.T, w2k.T, w1v.T, w2v.T)


def kernel(x, pos, edge_index, W_query, W_sim, W1k, W2k, W1v, W2v):
    src = edge_index[0]
    dst = edge_index[1]

    x_src, sq1 = _sc_gather(x, pos.reshape(-1), src, dst)

    logits, valuesT = _edge_compute(
        x_src, sq1, W_query, W_sim[:, :, 0], W1k, W2k, W1v, W2v)

    (m_p,) = _sc_segmax(src, logits)
    exh, s_p = _sc_segsum(src, logits, m_p)
    rs = _rsqrt_nodes(s_p)
    zeros = jnp.zeros((_SL, MUL), jnp.float32)
    (out_p,) = _sc_scatter_out(dst, src, valuesT, exh, rs, zeros)
    return _sum_halves(out_p)[:N_NODES]
